# baseline (speedup 1.0000x reference)
import functools

import jax
import jax.numpy as jnp
from jax import lax
from jax.experimental import pallas as pl
from jax.experimental.pallas import tpu as pltpu
from jax.experimental.pallas import tpu_sc as plsc

N = 1000
E = 1000
D = 256
NPAD = 1024
EPAD = 1024
ZROW = N
DUMP = NPAD - 1
NCORE = 2
EPC = EPAD // NCORE
EPT = EPC // 16
CW = 16

_SC_MESH = plsc.VectorSubcoreMesh(
    core_axis_name="c", subcore_axis_name="s", num_cores=NCORE
)
_SC_PARAMS = pltpu.CompilerParams(
    use_tc_tiling_on_sc=False, needs_layout_passes=False
)


@functools.partial(
    pl.kernel,
    out_type=jax.ShapeDtypeStruct((NPAD, CW), jnp.float32),
    scratch_types=[
        pltpu.VMEM((EPAD,), jnp.int32),
        pltpu.VMEM((NPAD, CW), jnp.float32),
    ],
    mesh=_SC_MESH,
    compiler_params=_SC_PARAMS,
)
def _degree_kernel(dst_hbm, zeros_hbm, deg_hbm, dsts_v, acc_v):
    cid = lax.axis_index("c")
    sid = lax.axis_index("s")
    iota = lax.iota(jnp.int32, 16)

    @pl.when(jnp.logical_and(cid == 0, sid == 0))
    def _():
        pltpu.sync_copy(dst_hbm, dsts_v)
        pltpu.sync_copy(zeros_hbm, acc_v)
        ones = jnp.ones((16,), jnp.float32)

        zvec = jnp.zeros((16,), jnp.int32)

        @plsc.parallel_loop(0, EPAD // 16, 1, unroll=4)
        def body(k_):
            dvec = dsts_v[pl.ds(k_ * 16, 16)]
            plsc.addupdate_scatter(acc_v, [dvec, zvec], ones)

        pltpu.sync_copy(acc_v, deg_hbm)


@functools.partial(
    pl.kernel,
    out_type=(
        jax.ShapeDtypeStruct((NCORE, NPAD, D), jnp.float32),
        jax.ShapeDtypeStruct((EPAD, D), jnp.float32),
    ),
    scratch_types=[
        pltpu.VMEM((EPT,), jnp.int32),
        pltpu.VMEM((EPC,), jnp.int32),
        pltpu.VMEM((EPT, D), jnp.float32),
        pltpu.VMEM((NPAD, CW), jnp.float32),
        pltpu.VMEM((EPC, CW), jnp.float32),
        pltpu.SemaphoreType.DMA,
    ],
    mesh=_SC_MESH,
    compiler_params=_SC_PARAMS,
)
def _agg_kernel(t_hbm, src_hbm, dst_hbm, zeros_hbm, out_hbm, g_hbm,
                idx_s, dsts_v, rows_v, acc_v, gc_v, sem):
    cid = lax.axis_index("c")
    sid = lax.axis_index("s")
    ebase = cid * EPC + sid * EPT
    c = sid * CW
    pltpu.sync_copy(src_hbm.at[pl.ds(ebase, EPT)], idx_s)
    gather = pltpu.async_copy(t_hbm.at[idx_s], rows_v, sem)
    pltpu.sync_copy(dst_hbm.at[pl.ds(cid * EPC, EPC)], dsts_v)

    @pl.when(cid == 0)
    def _():
        pltpu.sync_copy(t_hbm.at[slice(None), pl.ds(c, CW)], acc_v)

    @pl.when(cid != 0)
    def _():
        pltpu.sync_copy(zeros_hbm, acc_v)

    gather.wait()
    pltpu.sync_copy(rows_v, g_hbm.at[pl.ds(ebase, EPT)])
    plsc.subcore_barrier()
    pltpu.sync_copy(g_hbm.at[pl.ds(cid * EPC, EPC), pl.ds(c, CW)], gc_v)
    iota = lax.iota(jnp.int32, 16)

    @plsc.parallel_loop(0, EPC // 16, 1, unroll=2)
    def body(k_):
        dvec = dsts_v[pl.ds(k_ * 16, 16)]
        evec = jnp.full((16,), k_ * 16, jnp.int32) + iota
        for j in range(16):
            jvec = jnp.full((16,), j, jnp.int32)
            vals = plsc.load_gather(gc_v, [evec, jvec])
            plsc.addupdate_scatter(acc_v, [dvec, jvec], vals)

    pltpu.sync_copy(acc_v, out_hbm.at[cid, slice(None), pl.ds(c, CW)])


def _mm_body(x_ref, w_ref, o_ref):
    o_ref[...] = lax.dot_general(
        x_ref[...], w_ref[...],
        (((1,), (1,)), ((), ())),
        preferred_element_type=jnp.float32,
    )


def _mm_first(x_pad, W):
    return pl.pallas_call(
        _mm_body,
        out_shape=jax.ShapeDtypeStruct((NPAD, D), jnp.float32),
    )(x_pad, W)


def _mid_body(acc_ref, deg_ref, b_ref, w_ref, o_ref):
    cnt = deg_ref[:, 0:1] + 1.0
    acc = acc_ref[0, :, :] + acc_ref[1, :, :]
    y = acc / cnt + b_ref[...]
    rows = lax.broadcasted_iota(jnp.int32, (NPAD, D), 0)
    maskf = (rows < N).astype(jnp.float32)
    ym = y * maskf
    m = jnp.sum(ym, axis=0, keepdims=True) / float(N)
    dlt = y - m
    var = jnp.sum(dlt * dlt * maskf, axis=0, keepdims=True) / float(N)
    h = jnp.maximum(dlt / jnp.sqrt(var + 1e-5), 0.0) * maskf
    o_ref[...] = lax.dot_general(
        h, w_ref[...],
        (((1,), (1,)), ((), ())),
        preferred_element_type=jnp.float32,
    )


def _mid(acc, deg, b, W):
    return pl.pallas_call(
        _mid_body,
        out_shape=jax.ShapeDtypeStruct((NPAD, D), jnp.float32),
    )(acc, deg, b, W)


def _last_body(acc_ref, deg_ref, b_ref, o_ref):
    cnt = deg_ref[:N, 0:1] + 1.0
    acc = acc_ref[0, :N, :] + acc_ref[1, :N, :]
    o_ref[...] = acc / cnt + b_ref[...]


def _last(acc, deg, b):
    return pl.pallas_call(
        _last_body,
        out_shape=jax.ShapeDtypeStruct((N, D), jnp.float32),
    )(acc, deg, b)


def kernel(edge_index, x, W0, b0, W1, b1, W2, b2, W3, b3, W4, b4):
    src = edge_index[0]
    dst = edge_index[1]
    pad_s = jnp.full((EPAD - E,), ZROW, jnp.int32)
    pad_d = jnp.full((EPAD - E,), DUMP, jnp.int32)
    src_p = jnp.concatenate([src, pad_s])
    dst_p = jnp.concatenate([dst, pad_d])
    x_pad = jnp.zeros((NPAD, D), jnp.float32).at[:N].set(x)
    zeros16 = jnp.zeros((NPAD, CW), jnp.float32)

    deg = _degree_kernel(dst_p, zeros16)
    bias = [b.reshape(1, D) for b in (b0, b1, b2, b3, b4)]
    Ws = [W0, W1, W2, W3, W4]

    t = _mm_first(x_pad, Ws[0])
    for i in range(4):
        acc, _g = _agg_kernel(t, src_p, dst_p, zeros16)
        t = _mid(acc, deg, bias[i], Ws[i + 1])
    acc, _g = _agg_kernel(t, src_p, dst_p, zeros16)
    return _last(acc, deg, bias[4])


# speedup vs baseline: 1.1308x; 1.1308x over previous
import functools

import jax
import jax.numpy as jnp
from jax import lax
from jax.experimental import pallas as pl
from jax.experimental.pallas import tpu as pltpu
from jax.experimental.pallas import tpu_sc as plsc

N = 1000
E = 1000
D = 256
NPAD = 1024
EPAD = 1024
ZROW = N
DUMP = NPAD - 1
NCORE = 2
EPC = EPAD // NCORE
EPT = EPC // 16
CW = 16

_SC_MESH = plsc.VectorSubcoreMesh(
    core_axis_name="c", subcore_axis_name="s", num_cores=NCORE
)
_SC_PARAMS = pltpu.CompilerParams(
    use_tc_tiling_on_sc=False, needs_layout_passes=False
)


@functools.partial(
    pl.kernel,
    out_type=jax.ShapeDtypeStruct((NPAD, CW), jnp.float32),
    scratch_types=[
        pltpu.VMEM((EPAD,), jnp.int32),
        pltpu.VMEM((NPAD, CW), jnp.float32),
    ],
    mesh=_SC_MESH,
    compiler_params=_SC_PARAMS,
)
def _degree_kernel(dst_hbm, zeros_hbm, deg_hbm, dsts_v, acc_v):
    cid = lax.axis_index("c")
    sid = lax.axis_index("s")
    iota = lax.iota(jnp.int32, 16)

    @pl.when(jnp.logical_and(cid == 0, sid == 0))
    def _():
        pltpu.sync_copy(dst_hbm, dsts_v)
        pltpu.sync_copy(zeros_hbm, acc_v)
        ones = jnp.ones((16,), jnp.float32)

        @plsc.parallel_loop(0, EPAD // 16, 1, unroll=4)
        def body(k_):
            dvec = dsts_v[pl.ds(k_ * 16, 16)]
            for j in range(16):
                d = dvec[j]
                plsc.addupdate_scatter(
                    acc_v, [jnp.full((16,), d, jnp.int32), iota], ones
                )

        pltpu.sync_copy(acc_v, deg_hbm)


@functools.partial(
    pl.kernel,
    out_type=(
        jax.ShapeDtypeStruct((NCORE, NPAD, D), jnp.float32),
        jax.ShapeDtypeStruct((EPAD, D), jnp.float32),
    ),
    scratch_types=[
        pltpu.VMEM((EPT,), jnp.int32),
        pltpu.VMEM((EPC,), jnp.int32),
        pltpu.VMEM((EPT, D), jnp.float32),
        pltpu.VMEM((NPAD, CW), jnp.float32),
        pltpu.VMEM((EPC, CW), jnp.float32),
        pltpu.SemaphoreType.DMA,
    ],
    mesh=_SC_MESH,
    compiler_params=_SC_PARAMS,
)
def _agg_kernel(t_hbm, src_hbm, dst_hbm, zeros_hbm, out_hbm, g_hbm,
                idx_s, dsts_v, rows_v, acc_v, gc_v, sem):
    cid = lax.axis_index("c")
    sid = lax.axis_index("s")
    ebase = cid * EPC + sid * EPT
    c = sid * CW
    pltpu.sync_copy(src_hbm.at[pl.ds(ebase, EPT)], idx_s)
    gather = pltpu.async_copy(t_hbm.at[idx_s], rows_v, sem)
    pltpu.sync_copy(dst_hbm.at[pl.ds(cid * EPC, EPC)], dsts_v)

    @pl.when(cid == 0)
    def _():
        pltpu.sync_copy(t_hbm.at[slice(None), pl.ds(c, CW)], acc_v)

    @pl.when(cid != 0)
    def _():
        pltpu.sync_copy(zeros_hbm, acc_v)

    gather.wait()
    pltpu.sync_copy(rows_v, g_hbm.at[pl.ds(ebase, EPT)])
    plsc.subcore_barrier()
    pltpu.sync_copy(g_hbm.at[pl.ds(cid * EPC, EPC), pl.ds(c, CW)], gc_v)
    iota = lax.iota(jnp.int32, 16)

    @plsc.parallel_loop(0, EPC // 16, 1, unroll=4)
    def body(k_):
        dvec = dsts_v[pl.ds(k_ * 16, 16)]
        for j in range(16):
            d = dvec[j]
            row = plsc.load_gather(
                gc_v, [jnp.full((16,), k_ * 16 + j, jnp.int32), iota]
            )
            plsc.addupdate_scatter(
                acc_v, [jnp.full((16,), d, jnp.int32), iota], row
            )

    pltpu.sync_copy(acc_v, out_hbm.at[cid, slice(None), pl.ds(c, CW)])


def _mm_body(x_ref, w_ref, o_ref):
    o_ref[...] = lax.dot_general(
        x_ref[...], w_ref[...],
        (((1,), (1,)), ((), ())),
        preferred_element_type=jnp.float32,
    )


def _mm_first(x_pad, W):
    return pl.pallas_call(
        _mm_body,
        out_shape=jax.ShapeDtypeStruct((NPAD, D), jnp.float32),
    )(x_pad, W)


def _mid_body(acc_ref, deg_ref, b_ref, w_ref, o_ref):
    cnt = deg_ref[:, 0:1] + 1.0
    acc = acc_ref[0, :, :] + acc_ref[1, :, :]
    y = acc / cnt + b_ref[...]
    rows = lax.broadcasted_iota(jnp.int32, (NPAD, D), 0)
    maskf = (rows < N).astype(jnp.float32)
    ym = y * maskf
    m = jnp.sum(ym, axis=0, keepdims=True) / float(N)
    dlt = y - m
    var = jnp.sum(dlt * dlt * maskf, axis=0, keepdims=True) / float(N)
    h = jnp.maximum(dlt / jnp.sqrt(var + 1e-5), 0.0) * maskf
    o_ref[...] = lax.dot_general(
        h, w_ref[...],
        (((1,), (1,)), ((), ())),
        preferred_element_type=jnp.float32,
    )


def _mid(acc, deg, b, W):
    return pl.pallas_call(
        _mid_body,
        out_shape=jax.ShapeDtypeStruct((NPAD, D), jnp.float32),
    )(acc, deg, b, W)


def _last_body(acc_ref, deg_ref, b_ref, o_ref):
    cnt = deg_ref[:N, 0:1] + 1.0
    acc = acc_ref[0, :N, :] + acc_ref[1, :N, :]
    o_ref[...] = acc / cnt + b_ref[...]


def _last(acc, deg, b):
    return pl.pallas_call(
        _last_body,
        out_shape=jax.ShapeDtypeStruct((N, D), jnp.float32),
    )(acc, deg, b)


def kernel(edge_index, x, W0, b0, W1, b1, W2, b2, W3, b3, W4, b4):
    src = edge_index[0]
    dst = edge_index[1]
    pad_s = jnp.full((EPAD - E,), ZROW, jnp.int32)
    pad_d = jnp.full((EPAD - E,), DUMP, jnp.int32)
    src_p = jnp.concatenate([src, pad_s])
    dst_p = jnp.concatenate([dst, pad_d])
    x_pad = jnp.zeros((NPAD, D), jnp.float32).at[:N].set(x)
    zeros16 = jnp.zeros((NPAD, CW), jnp.float32)

    deg = _degree_kernel(dst_p, zeros16)
    bias = [b.reshape(1, D) for b in (b0, b1, b2, b3, b4)]
    Ws = [W0, W1, W2, W3, W4]

    t = _mm_first(x_pad, Ws[0])
    for i in range(4):
        acc, _g = _agg_kernel(t, src_p, dst_p, zeros16)
        t = _mid(acc, deg, bias[i], Ws[i + 1])
    acc, _g = _agg_kernel(t, src_p, dst_p, zeros16)
    return _last(acc, deg, bias[4])


# speedup vs baseline: 1.1354x; 1.0041x over previous
import functools

import jax
import jax.numpy as jnp
from jax import lax
from jax.experimental import pallas as pl
from jax.experimental.pallas import tpu as pltpu
from jax.experimental.pallas import tpu_sc as plsc

N = 1000
E = 1000
D = 256
NPAD = 1024
EPAD = 1024
ZROW = N
DUMP = NPAD - 1
NCORE = 2
EPC = EPAD // NCORE
EPT = EPC // 16
CW = 16

_SC_MESH = plsc.VectorSubcoreMesh(
    core_axis_name="c", subcore_axis_name="s", num_cores=NCORE
)
_SC_PARAMS = pltpu.CompilerParams(
    use_tc_tiling_on_sc=False, needs_layout_passes=False
)


@functools.partial(
    pl.kernel,
    out_type=jax.ShapeDtypeStruct((NPAD, CW), jnp.float32),
    scratch_types=[
        pltpu.VMEM((EPAD,), jnp.int32),
        pltpu.VMEM((NPAD, CW), jnp.float32),
    ],
    mesh=_SC_MESH,
    compiler_params=_SC_PARAMS,
)
def _degree_kernel(dst_hbm, zeros_hbm, deg_hbm, dsts_v, acc_v):
    cid = lax.axis_index("c")
    sid = lax.axis_index("s")
    iota = lax.iota(jnp.int32, 16)

    @pl.when(jnp.logical_and(cid == 0, sid == 0))
    def _():
        pltpu.sync_copy(dst_hbm, dsts_v)
        pltpu.sync_copy(zeros_hbm, acc_v)
        ones = jnp.ones((16,), jnp.float32)

        @plsc.parallel_loop(0, EPAD // 16, 1, unroll=4)
        def body(k_):
            dvec = dsts_v[pl.ds(k_ * 16, 16)]
            for j in range(16):
                d = dvec[j]
                plsc.addupdate_scatter(
                    acc_v, [jnp.full((16,), d, jnp.int32), iota], ones
                )

        pltpu.sync_copy(acc_v, deg_hbm)


@functools.partial(
    pl.kernel,
    out_type=(
        jax.ShapeDtypeStruct((NCORE, NPAD, D), jnp.float32),
        jax.ShapeDtypeStruct((EPAD, D), jnp.float32),
    ),
    scratch_types=[
        pltpu.VMEM((EPT,), jnp.int32),
        pltpu.VMEM((EPC,), jnp.int32),
        pltpu.VMEM((EPT, D), jnp.float32),
        pltpu.VMEM((NPAD, CW), jnp.float32),
        pltpu.VMEM((EPC, CW), jnp.float32),
        pltpu.SemaphoreType.DMA,
    ],
    mesh=_SC_MESH,
    compiler_params=_SC_PARAMS,
)
def _agg_kernel(t_hbm, src_hbm, dst_hbm, zeros_hbm, out_hbm, g_hbm,
                idx_s, dsts_v, rows_v, acc_v, gc_v, sem):
    cid = lax.axis_index("c")
    sid = lax.axis_index("s")
    ebase = cid * EPC + sid * EPT
    c = sid * CW
    pltpu.sync_copy(src_hbm.at[pl.ds(ebase, EPT)], idx_s)
    gather = pltpu.async_copy(t_hbm.at[idx_s], rows_v, sem)
    pltpu.sync_copy(dst_hbm.at[pl.ds(cid * EPC, EPC)], dsts_v)

    @pl.when(cid == 0)
    def _():
        pltpu.sync_copy(t_hbm.at[slice(None), pl.ds(c, CW)], acc_v)

    @pl.when(cid != 0)
    def _():
        pltpu.sync_copy(zeros_hbm, acc_v)

    gather.wait()
    pltpu.sync_copy(rows_v, g_hbm.at[pl.ds(ebase, EPT)])
    plsc.subcore_barrier()
    pltpu.sync_copy(g_hbm.at[pl.ds(cid * EPC, EPC), pl.ds(c, CW)], gc_v)
    iota = lax.iota(jnp.int32, 16)

    @plsc.parallel_loop(0, EPC // 16, 1, unroll=8)
    def body(k_):
        dvec = dsts_v[pl.ds(k_ * 16, 16)]
        for j in range(16):
            d = dvec[j]
            row = plsc.load_gather(
                gc_v, [jnp.full((16,), k_ * 16 + j, jnp.int32), iota]
            )
            plsc.addupdate_scatter(
                acc_v, [jnp.full((16,), d, jnp.int32), iota], row
            )

    pltpu.sync_copy(acc_v, out_hbm.at[cid, slice(None), pl.ds(c, CW)])


def _mm_body(x_ref, w_ref, o_ref):
    o_ref[...] = lax.dot_general(
        x_ref[...], w_ref[...],
        (((1,), (1,)), ((), ())),
        preferred_element_type=jnp.float32,
    )


def _mm_first(x_pad, W):
    return pl.pallas_call(
        _mm_body,
        out_shape=jax.ShapeDtypeStruct((NPAD, D), jnp.float32),
    )(x_pad, W)


def _mid_body(acc_ref, deg_ref, b_ref, w_ref, o_ref):
    cnt = deg_ref[:, 0:1] + 1.0
    acc = acc_ref[0, :, :] + acc_ref[1, :, :]
    y = acc / cnt + b_ref[...]
    rows = lax.broadcasted_iota(jnp.int32, (NPAD, D), 0)
    maskf = (rows < N).astype(jnp.float32)
    ym = y * maskf
    m = jnp.sum(ym, axis=0, keepdims=True) / float(N)
    dlt = y - m
    var = jnp.sum(dlt * dlt * maskf, axis=0, keepdims=True) / float(N)
    h = jnp.maximum(dlt / jnp.sqrt(var + 1e-5), 0.0) * maskf
    o_ref[...] = lax.dot_general(
        h, w_ref[...],
        (((1,), (1,)), ((), ())),
        preferred_element_type=jnp.float32,
    )


def _mid(acc, deg, b, W):
    return pl.pallas_call(
        _mid_body,
        out_shape=jax.ShapeDtypeStruct((NPAD, D), jnp.float32),
    )(acc, deg, b, W)


def _last_body(acc_ref, deg_ref, b_ref, o_ref):
    cnt = deg_ref[:N, 0:1] + 1.0
    acc = acc_ref[0, :N, :] + acc_ref[1, :N, :]
    o_ref[...] = acc / cnt + b_ref[...]


def _last(acc, deg, b):
    return pl.pallas_call(
        _last_body,
        out_shape=jax.ShapeDtypeStruct((N, D), jnp.float32),
    )(acc, deg, b)


def kernel(edge_index, x, W0, b0, W1, b1, W2, b2, W3, b3, W4, b4):
    src = edge_index[0]
    dst = edge_index[1]
    pad_s = jnp.full((EPAD - E,), ZROW, jnp.int32)
    pad_d = jnp.full((EPAD - E,), DUMP, jnp.int32)
    src_p = jnp.concatenate([src, pad_s])
    dst_p = jnp.concatenate([dst, pad_d])
    x_pad = jnp.zeros((NPAD, D), jnp.float32).at[:N].set(x)
    zeros16 = jnp.zeros((NPAD, CW), jnp.float32)

    deg = _degree_kernel(dst_p, zeros16)
    bias = [b.reshape(1, D) for b in (b0, b1, b2, b3, b4)]
    Ws = [W0, W1, W2, W3, W4]

    t = _mm_first(x_pad, Ws[0])
    for i in range(4):
        acc, _g = _agg_kernel(t, src_p, dst_p, zeros16)
        t = _mid(acc, deg, bias[i], Ws[i + 1])
    acc, _g = _agg_kernel(t, src_p, dst_p, zeros16)
    return _last(acc, deg, bias[4])
